# Initial kernel scaffold; baseline (speedup 1.0000x reference)
import jax
import jax.numpy as jnp
from jax.experimental import pallas as pl


def kernel(edge_index, attr_mtx, input_embed, W0, W1):
    raise NotImplementedError("write your pallas kernel here")


# speedup vs baseline: 11.1607x; 11.1607x over previous
import functools

import jax
import jax.numpy as jnp
from jax import lax
from jax.experimental import pallas as pl
from jax.experimental.pallas import tpu as pltpu
from jax.experimental.pallas import tpu_sc as plsc

N = 10000
E = 320000
D = 128
A = 16
LDA = 0.05

NC = 2
NS = 16
NW = NC * NS
CK = 128
NCHUNK = -(-E // (NW * CK))
EPW = NCHUNK * CK
EPAD = NW * EPW
NCC = NCHUNK
NPAD = NCC * CK
MAXC = -(-NCC // NS)

_sc_mesh = plsc.VectorSubcoreMesh(
    core_axis_name="c", subcore_axis_name="s", num_cores=NC, num_subcores=NS
)


@functools.partial(
    pl.kernel,
    out_type=jax.ShapeDtypeStruct((NC, NPAD, D), jnp.float32),
    mesh=_sc_mesh,
    scratch_types=[
        pltpu.VMEM_SHARED((NPAD, D), jnp.float32),
        pltpu.VMEM((2, 2, CK), jnp.int32),
        pltpu.VMEM((CK, D), jnp.float32),
        pltpu.VMEM((CK, D), jnp.float32),
        pltpu.SemaphoreType.DMA,
        pltpu.SemaphoreType.DMA,
    ],
)
def _deg_kernel(idx3, cst, out, deg_sh, crb, ones, zb, si0, si1):
    cid = lax.axis_index("c")
    sid = lax.axis_index("s")
    w = cid * NS + sid

    pltpu.sync_copy(cst.at[0], zb)
    pltpu.sync_copy(cst.at[1], ones)
    for i in range(MAXC):
        ci = sid + NS * i

        @pl.when(ci < NCC)
        def _():
            off = pl.multiple_of(ci * CK, 8)
            pltpu.sync_copy(zb, deg_sh.at[pl.ds(off, CK)])

    plsc.subcore_barrier()

    pltpu.sync_copy(idx3.at[w, 0], crb.at[0])
    pltpu.async_copy(idx3.at[w, 1], crb.at[1], si1)

    def body(g, carry):
        j0 = 2 * g
        pltpu.sync_copy(ones, deg_sh.at[crb.at[0, 1]], add=True)
        pltpu.async_copy(idx3.at[w, j0 + 2], crb.at[0], si0)
        pltpu.make_async_copy(idx3.at[w, j0 + 1], crb.at[1], si1).wait()
        pltpu.sync_copy(ones, deg_sh.at[crb.at[1, 1]], add=True)
        jn = jnp.minimum(j0 + 3, NCHUNK - 1)
        pltpu.async_copy(idx3.at[w, jn], crb.at[1], si1)
        pltpu.make_async_copy(idx3.at[w, j0 + 2], crb.at[0], si0).wait()
        return carry

    lax.fori_loop(0, NCHUNK // 2, body, 0)
    pltpu.sync_copy(ones, deg_sh.at[crb.at[0, 1]], add=True)
    pltpu.make_async_copy(idx3.at[w, NCHUNK - 1], crb.at[1], si1).wait()

    plsc.subcore_barrier()
    for i in range(MAXC):
        ci = sid + NS * i

        @pl.when(ci < NCC)
        def _():
            off = pl.multiple_of(ci * CK, 8)
            pltpu.sync_copy(deg_sh.at[pl.ds(off, CK)], out.at[cid, pl.ds(off, CK)])


@functools.partial(
    pl.kernel,
    out_type=jax.ShapeDtypeStruct((NC, NPAD, D), jnp.float32),
    mesh=_sc_mesh,
    scratch_types=[
        pltpu.VMEM_SHARED((NPAD, D), jnp.float32),
        pltpu.VMEM((2, 2, CK), jnp.int32),
        pltpu.VMEM((CK, D), jnp.float32),
        pltpu.VMEM((CK, D), jnp.float32),
        pltpu.SemaphoreType.DMA,
        pltpu.SemaphoreType.DMA,
        pltpu.SemaphoreType.DMA,
        pltpu.SemaphoreType.DMA,
    ],
)
def _spmm_kernel(zs, idx3, out, agg_sh, crb, b0, b1, si0, si1, s0, s1):
    cid = lax.axis_index("c")
    sid = lax.axis_index("s")
    w = cid * NS + sid

    def fill(i, carry):
        for k in range(D // 16):
            b0[i, pl.ds(k * 16, 16)] = jnp.zeros((16,), jnp.float32)
        return carry

    lax.fori_loop(0, CK, fill, 0)
    for i in range(MAXC):
        ci = sid + NS * i

        @pl.when(ci < NCC)
        def _():
            off = pl.multiple_of(ci * CK, 8)
            pltpu.sync_copy(b0, agg_sh.at[pl.ds(off, CK)])

    plsc.subcore_barrier()

    pltpu.sync_copy(idx3.at[w, 0], crb.at[0])
    pltpu.async_copy(idx3.at[w, 1], crb.at[1], si1)
    pltpu.async_copy(zs.at[crb.at[0, 0]], b0, s0)

    def body(g, carry):
        j0 = 2 * g
        pltpu.make_async_copy(idx3.at[w, j0 + 1], crb.at[1], si1).wait()
        pltpu.async_copy(zs.at[crb.at[1, 0]], b1, s1)
        pltpu.make_async_copy(zs.at[crb.at[0, 0]], b0, s0).wait()
        pltpu.sync_copy(b0, agg_sh.at[crb.at[0, 1]], add=True)
        pltpu.async_copy(idx3.at[w, j0 + 2], crb.at[0], si0)
        pltpu.make_async_copy(zs.at[crb.at[1, 0]], b1, s1).wait()
        pltpu.sync_copy(b1, agg_sh.at[crb.at[1, 1]], add=True)
        jn = jnp.minimum(j0 + 3, NCHUNK - 1)
        pltpu.async_copy(idx3.at[w, jn], crb.at[1], si1)
        pltpu.make_async_copy(idx3.at[w, j0 + 2], crb.at[0], si0).wait()
        pltpu.async_copy(zs.at[crb.at[0, 0]], b0, s0)
        return carry

    lax.fori_loop(0, NCHUNK // 2, body, 0)
    pltpu.make_async_copy(zs.at[crb.at[0, 0]], b0, s0).wait()
    pltpu.sync_copy(b0, agg_sh.at[crb.at[0, 1]], add=True)
    pltpu.make_async_copy(idx3.at[w, NCHUNK - 1], crb.at[1], si1).wait()

    plsc.subcore_barrier()
    for i in range(MAXC):
        ci = sid + NS * i

        @pl.when(ci < NCC)
        def _():
            off = pl.multiple_of(ci * CK, 8)
            pltpu.sync_copy(agg_sh.at[pl.ds(off, CK)], out.at[cid, pl.ds(off, CK)])


BN = 1264
GRID = NPAD // BN
F32 = jnp.float32


def _prep_body(dp, x, at, w0a, w0b, z, zsc, dinv, selfw):
    deg = dp[0, :, 0:1] + dp[1, :, 0:1]
    rs = deg * (1.0 + LDA)
    pos = rs > 0.0
    dis = jnp.where(pos, lax.rsqrt(rs), 0.0)
    sw = jnp.where(pos, (deg * LDA) / rs, 0.0)
    zv = jnp.dot(x[...], w0a[...], preferred_element_type=F32)
    zv = zv + jnp.dot(at[...], w0b[...], preferred_element_type=F32)
    z[...] = zv
    zsc[...] = zv * dis
    dinv[...] = dis
    selfw[...] = sw


_prep = pl.pallas_call(
    _prep_body,
    grid=(GRID,),
    in_specs=[
        pl.BlockSpec((2, BN, D), lambda i: (0, i, 0)),
        pl.BlockSpec((BN, D), lambda i: (i, 0)),
        pl.BlockSpec((BN, A), lambda i: (i, 0)),
        pl.BlockSpec((D, D), lambda i: (0, 0)),
        pl.BlockSpec((A, D), lambda i: (0, 0)),
    ],
    out_specs=[
        pl.BlockSpec((BN, D), lambda i: (i, 0)),
        pl.BlockSpec((BN, D), lambda i: (i, 0)),
        pl.BlockSpec((BN, 1), lambda i: (i, 0)),
        pl.BlockSpec((BN, 1), lambda i: (i, 0)),
    ],
    out_shape=[
        jax.ShapeDtypeStruct((NPAD, D), F32),
        jax.ShapeDtypeStruct((NPAD, D), F32),
        jax.ShapeDtypeStruct((NPAD, 1), F32),
        jax.ShapeDtypeStruct((NPAD, 1), F32),
    ],
)


def _mid_body(p, z1, dinv, selfw, at, w1a, w1b, z2, zs2):
    h = jnp.tanh(dinv[...] * (p[0] + p[1]) + selfw[...] * z1[...])
    zv = jnp.dot(h, w1a[...], preferred_element_type=F32)
    zv = zv + jnp.dot(at[...], w1b[...], preferred_element_type=F32)
    z2[...] = zv
    zs2[...] = zv * dinv[...]


_mid = pl.pallas_call(
    _mid_body,
    grid=(GRID,),
    in_specs=[
        pl.BlockSpec((2, BN, D), lambda i: (0, i, 0)),
        pl.BlockSpec((BN, D), lambda i: (i, 0)),
        pl.BlockSpec((BN, 1), lambda i: (i, 0)),
        pl.BlockSpec((BN, 1), lambda i: (i, 0)),
        pl.BlockSpec((BN, A), lambda i: (i, 0)),
        pl.BlockSpec((D, D), lambda i: (0, 0)),
        pl.BlockSpec((A, D), lambda i: (0, 0)),
    ],
    out_specs=[
        pl.BlockSpec((BN, D), lambda i: (i, 0)),
        pl.BlockSpec((BN, D), lambda i: (i, 0)),
    ],
    out_shape=[
        jax.ShapeDtypeStruct((NPAD, D), F32),
        jax.ShapeDtypeStruct((NPAD, D), F32),
    ],
)


def _fin_body(p, z2, dinv, selfw, out):
    h = jnp.tanh(dinv[...] * (p[0] + p[1]) + selfw[...] * z2[...])
    ss = jnp.sum(h * h, axis=1, keepdims=True)
    out[...] = h * lax.rsqrt(jnp.maximum(ss, 1e-12))


_fin = pl.pallas_call(
    _fin_body,
    grid=(GRID,),
    in_specs=[
        pl.BlockSpec((2, BN, D), lambda i: (0, i, 0)),
        pl.BlockSpec((BN, D), lambda i: (i, 0)),
        pl.BlockSpec((BN, 1), lambda i: (i, 0)),
        pl.BlockSpec((BN, 1), lambda i: (i, 0)),
    ],
    out_specs=pl.BlockSpec((BN, D), lambda i: (i, 0)),
    out_shape=jax.ShapeDtypeStruct((NPAD, D), F32),
)


def kernel(edge_index, attr_mtx, input_embed, W0, W1):
    pad = EPAD - E
    colp = jnp.concatenate([edge_index[1], jnp.full((pad,), N, jnp.int32)])
    rowp = jnp.concatenate([edge_index[0], jnp.full((pad,), N, jnp.int32)])
    idx3 = jnp.stack(
        [colp.reshape(NW, NCHUNK, CK), rowp.reshape(NW, NCHUNK, CK)], axis=2
    )
    xp = jnp.pad(input_embed, ((0, NPAD - N), (0, 0)))
    atp = jnp.pad(attr_mtx, ((0, NPAD - N), (0, 0)))
    w0a, w0b = W0[:D], W0[D:]
    w1a, w1b = W1[:D], W1[D:]

    cst = jnp.stack([jnp.zeros((CK, D), jnp.float32), jnp.ones((CK, D), jnp.float32)])
    degp = _deg_kernel(idx3, cst)
    z1, zs1, dinv, selfw = _prep(degp, xp, atp, w0a, w0b)
    p1 = _spmm_kernel(zs1, idx3)
    z2, zs2 = _mid(p1, z1, dinv, selfw, atp, w1a, w1b)
    p2 = _spmm_kernel(zs2, idx3)
    out = _fin(p2, z2, dinv, selfw)
    return out[:N]


# speedup vs baseline: 14.0879x; 1.2623x over previous
import functools

import jax
import jax.numpy as jnp
from jax import lax
from jax.experimental import pallas as pl
from jax.experimental.pallas import tpu as pltpu
from jax.experimental.pallas import tpu_sc as plsc

N = 10000
E = 320000
D = 128
A = 16
LDA = 0.05

NC = 2
NS = 16
NW = NC * NS
CK = 112
NCHUNK = -(-E // (NW * CK))
EPW = NCHUNK * CK
EPAD = NW * EPW
NCC = -(-(N + 1) // CK)
NPAD = NCC * CK
MAXC = -(-NCC // NS)

assert NCHUNK % 2 == 0 and NCHUNK >= 6 and (NCHUNK - 3) % 3 == 0

_sc_mesh = plsc.VectorSubcoreMesh(
    core_axis_name="c", subcore_axis_name="s", num_cores=NC, num_subcores=NS
)


def _make_deg_kernel(dw):
    @functools.partial(
        pl.kernel,
        out_type=jax.ShapeDtypeStruct((NC, NPAD, dw), jnp.float32),
        mesh=_sc_mesh,
        scratch_types=[
            pltpu.VMEM_SHARED((NPAD, dw), jnp.float32),
            pltpu.VMEM((2, 2, CK), jnp.int32),
            pltpu.VMEM((CK, dw), jnp.float32),
            pltpu.VMEM((CK, dw), jnp.float32),
            pltpu.SemaphoreType.DMA,
            pltpu.SemaphoreType.DMA,
        ],
    )
    def deg_kernel(idx3, cst, out, deg_sh, crb, ones, zb, si0, si1):
        cid = lax.axis_index("c")
        sid = lax.axis_index("s")
        w = cid * NS + sid

        pltpu.sync_copy(cst.at[0], zb)
        pltpu.sync_copy(cst.at[1], ones)
        for i in range(MAXC):
            ci = sid + NS * i

            @pl.when(ci < NCC)
            def _():
                off = pl.multiple_of(ci * CK, 8)
                pltpu.sync_copy(zb, deg_sh.at[pl.ds(off, CK)])

        plsc.subcore_barrier()

        pltpu.sync_copy(idx3.at[w, 0], crb.at[0])
        pltpu.async_copy(idx3.at[w, 1], crb.at[1], si1)

        def body(g, carry):
            j0 = 2 * g
            pltpu.sync_copy(ones, deg_sh.at[crb.at[0, 1]], add=True)
            jn0 = jnp.minimum(j0 + 2, NCHUNK - 1)
            pltpu.async_copy(idx3.at[w, jn0], crb.at[0], si0)
            pltpu.make_async_copy(idx3.at[w, j0 + 1], crb.at[1], si1).wait()
            pltpu.sync_copy(ones, deg_sh.at[crb.at[1, 1]], add=True)
            jn1 = jnp.minimum(j0 + 3, NCHUNK - 1)
            pltpu.async_copy(idx3.at[w, jn1], crb.at[1], si1)
            pltpu.make_async_copy(idx3.at[w, jn0], crb.at[0], si0).wait()
            return carry

        lax.fori_loop(0, NCHUNK // 2, body, 0)
        pltpu.make_async_copy(idx3.at[w, NCHUNK - 1], crb.at[1], si1).wait()

        plsc.subcore_barrier()
        for i in range(MAXC):
            ci = sid + NS * i

            @pl.when(ci < NCC)
            def _():
                off = pl.multiple_of(ci * CK, 8)
                pltpu.sync_copy(
                    deg_sh.at[pl.ds(off, CK)], out.at[cid, pl.ds(off, CK)]
                )

    return deg_kernel


_deg_kernel = _make_deg_kernel(D)


@functools.partial(
    pl.kernel,
    out_type=jax.ShapeDtypeStruct((NC, NPAD, D), jnp.float32),
    mesh=_sc_mesh,
    scratch_types=[
        pltpu.VMEM_SHARED((NPAD, D), jnp.float32),
        pltpu.VMEM((3, 2, CK), jnp.int32),
        pltpu.VMEM((CK, D), jnp.float32),
        pltpu.VMEM((CK, D), jnp.float32),
        pltpu.VMEM((CK, D), jnp.float32),
        pltpu.SemaphoreType.DMA,
        pltpu.SemaphoreType.DMA,
        pltpu.SemaphoreType.DMA,
        pltpu.SemaphoreType.DMA,
        pltpu.SemaphoreType.DMA,
        pltpu.SemaphoreType.DMA,
        pltpu.SemaphoreType.DMA,
        pltpu.SemaphoreType.DMA,
        pltpu.SemaphoreType.DMA,
    ],
)
def _spmm_kernel(zs, idx3, out, agg_sh, crb, b0, b1, b2,
                 i0, i1, i2, g0, g1, g2, s0, s1, s2):
    cid = lax.axis_index("c")
    sid = lax.axis_index("s")
    w = cid * NS + sid
    bufs = (b0, b1, b2)
    isem = (i0, i1, i2)
    gsem = (g0, g1, g2)
    ssem = (s0, s1, s2)

    def fill(i, carry):
        for k in range(D // 16):
            b0[i, pl.ds(k * 16, 16)] = jnp.zeros((16,), jnp.float32)
        return carry

    lax.fori_loop(0, CK, fill, 0)
    for i in range(MAXC):
        ci = sid + NS * i

        @pl.when(ci < NCC)
        def _():
            off = pl.multiple_of(ci * CK, 8)
            pltpu.sync_copy(b0, agg_sh.at[pl.ds(off, CK)])

    plsc.subcore_barrier()

    def idx_load(j, p, sem):
        pltpu.async_copy(idx3.at[w, j], crb.at[p], sem)

    def idx_wait(j, p, sem):
        pltpu.make_async_copy(idx3.at[w, j], crb.at[p], sem).wait()

    def gather(p):
        pltpu.async_copy(zs.at[crb.at[p, 0]], bufs[p], gsem[p])

    def gather_wait(p):
        pltpu.make_async_copy(zs.at[crb.at[p, 0]], bufs[p], gsem[p]).wait()

    def scatter(p):
        pltpu.async_copy(bufs[p], agg_sh.at[crb.at[p, 1]], ssem[p], add=True)

    def scatter_wait(p):
        pltpu.make_async_copy(bufs[p], agg_sh.at[crb.at[p, 1]], ssem[p]).wait()

    pltpu.sync_copy(idx3.at[w, 0], crb.at[0])
    gather(0)
    idx_load(1, 1, isem[1])
    idx_wait(1, 1, isem[1])
    gather(1)
    idx_load(2, 2, isem[2])
    gather_wait(0)
    scatter(0)
    idx_wait(2, 2, isem[2])
    gather(2)
    scatter_wait(0)
    idx_load(3, 0, isem[0])
    gather_wait(1)
    scatter(1)

    def body(g, carry):
        jb = 3 * g + 3
        for c in range(3):
            j = jb + c
            p = c
            pm = (c + 2) % 3
            pn = (c + 1) % 3
            idx_wait(j, p, isem[p])
            gather(p)

            @pl.when(j + 1 < NCHUNK)
            def _():
                scatter_wait(pn)
                idx_load(j + 1, pn, isem[pn])

            gather_wait(pm)
            scatter(pm)
        return carry

    lax.fori_loop(0, (NCHUNK - 3) // 3, body, 0)

    lastp = (NCHUNK - 1) % 3
    gather_wait(lastp)
    scatter(lastp)
    scatter_wait((NCHUNK - 3) % 3)
    scatter_wait((NCHUNK - 2) % 3)
    scatter_wait(lastp)

    plsc.subcore_barrier()
    for i in range(MAXC):
        ci = sid + NS * i

        @pl.when(ci < NCC)
        def _():
            off = pl.multiple_of(ci * CK, 8)
            pltpu.sync_copy(agg_sh.at[pl.ds(off, CK)], out.at[cid, pl.ds(off, CK)])


BN = 1680
GRID = NPAD // BN
F32 = jnp.float32


def _prep_body(dp, x, at, w0a, w0b, z, zsc, dinv, selfw):
    deg = dp[0, :, 0:1] + dp[1, :, 0:1]
    rs = deg * (1.0 + LDA)
    pos = rs > 0.0
    dis = jnp.where(pos, lax.rsqrt(rs), 0.0)
    sw = jnp.where(pos, (deg * LDA) / rs, 0.0)
    zv = jnp.dot(x[...], w0a[...], preferred_element_type=F32)
    zv = zv + jnp.dot(at[...], w0b[...], preferred_element_type=F32)
    z[...] = zv
    zsc[...] = zv * dis
    dinv[...] = dis
    selfw[...] = sw


_prep = pl.pallas_call(
    _prep_body,
    grid=(GRID,),
    in_specs=[
        pl.BlockSpec((2, BN, D), lambda i: (0, i, 0)),
        pl.BlockSpec((BN, D), lambda i: (i, 0)),
        pl.BlockSpec((BN, A), lambda i: (i, 0)),
        pl.BlockSpec((D, D), lambda i: (0, 0)),
        pl.BlockSpec((A, D), lambda i: (0, 0)),
    ],
    out_specs=[
        pl.BlockSpec((BN, D), lambda i: (i, 0)),
        pl.BlockSpec((BN, D), lambda i: (i, 0)),
        pl.BlockSpec((BN, 1), lambda i: (i, 0)),
        pl.BlockSpec((BN, 1), lambda i: (i, 0)),
    ],
    out_shape=[
        jax.ShapeDtypeStruct((NPAD, D), F32),
        jax.ShapeDtypeStruct((NPAD, D), F32),
        jax.ShapeDtypeStruct((NPAD, 1), F32),
        jax.ShapeDtypeStruct((NPAD, 1), F32),
    ],
)


def _mid_body(p, z1, dinv, selfw, at, w1a, w1b, z2, zs2):
    h = jnp.tanh(dinv[...] * (p[0] + p[1]) + selfw[...] * z1[...])
    zv = jnp.dot(h, w1a[...], preferred_element_type=F32)
    zv = zv + jnp.dot(at[...], w1b[...], preferred_element_type=F32)
    z2[...] = zv
    zs2[...] = zv * dinv[...]


_mid = pl.pallas_call(
    _mid_body,
    grid=(GRID,),
    in_specs=[
        pl.BlockSpec((2, BN, D), lambda i: (0, i, 0)),
        pl.BlockSpec((BN, D), lambda i: (i, 0)),
        pl.BlockSpec((BN, 1), lambda i: (i, 0)),
        pl.BlockSpec((BN, 1), lambda i: (i, 0)),
        pl.BlockSpec((BN, A), lambda i: (i, 0)),
        pl.BlockSpec((D, D), lambda i: (0, 0)),
        pl.BlockSpec((A, D), lambda i: (0, 0)),
    ],
    out_specs=[
        pl.BlockSpec((BN, D), lambda i: (i, 0)),
        pl.BlockSpec((BN, D), lambda i: (i, 0)),
    ],
    out_shape=[
        jax.ShapeDtypeStruct((NPAD, D), F32),
        jax.ShapeDtypeStruct((NPAD, D), F32),
    ],
)


def _fin_body(p, z2, dinv, selfw, out):
    h = jnp.tanh(dinv[...] * (p[0] + p[1]) + selfw[...] * z2[...])
    ss = jnp.sum(h * h, axis=1, keepdims=True)
    out[...] = h * lax.rsqrt(jnp.maximum(ss, 1e-12))


_fin = pl.pallas_call(
    _fin_body,
    grid=(GRID,),
    in_specs=[
        pl.BlockSpec((2, BN, D), lambda i: (0, i, 0)),
        pl.BlockSpec((BN, D), lambda i: (i, 0)),
        pl.BlockSpec((BN, 1), lambda i: (i, 0)),
        pl.BlockSpec((BN, 1), lambda i: (i, 0)),
    ],
    out_specs=pl.BlockSpec((BN, D), lambda i: (i, 0)),
    out_shape=jax.ShapeDtypeStruct((NPAD, D), F32),
)


def kernel(edge_index, attr_mtx, input_embed, W0, W1):
    pad = EPAD - E
    colp = jnp.concatenate([edge_index[1], jnp.full((pad,), N, jnp.int32)])
    rowp = jnp.concatenate([edge_index[0], jnp.full((pad,), N, jnp.int32)])
    idx3 = jnp.stack(
        [colp.reshape(NW, NCHUNK, CK), rowp.reshape(NW, NCHUNK, CK)], axis=2
    )
    xp = jnp.pad(input_embed, ((0, NPAD - N), (0, 0)))
    atp = jnp.pad(attr_mtx, ((0, NPAD - N), (0, 0)))
    w0a, w0b = W0[:D], W0[D:]
    w1a, w1b = W1[:D], W1[D:]

    cst = jnp.stack([jnp.zeros((CK, D), jnp.float32), jnp.ones((CK, D), jnp.float32)])
    degp = _deg_kernel(idx3, cst)
    z1, zs1, dinv, selfw = _prep(degp, xp, atp, w0a, w0b)
    p1 = _spmm_kernel(zs1, idx3)
    z2, zs2 = _mid(p1, z1, dinv, selfw, atp, w1a, w1b)
    p2 = _spmm_kernel(zs2, idx3)
    out = _fin(p2, z2, dinv, selfw)
    return out[:N]
